# baseline (speedup 1.0000x reference)
import functools

import jax
import jax.numpy as jnp
from jax import lax
from jax.experimental import pallas as pl
from jax.experimental.pallas import tpu as pltpu
from jax.experimental.pallas import tpu_sc as plsc

VOCAB = 100000
CONTEXT = 5
EMB = 64
BATCH = 1024
HIDDEN = 128
NLOOK = BATCH * 2 * CONTEXT


_NC = 2
_NS = 16
_NW = _NC * _NS
_BPW = NLOOK // _NW

@functools.cache
def _sc_gather_fn():
    mesh = plsc.VectorSubcoreMesh(core_axis_name="c", subcore_axis_name="s")

    @functools.partial(
        pl.kernel,
        mesh=mesh,
        out_type=jax.ShapeDtypeStruct((NLOOK, EMB), jnp.float32),
        scratch_types=[
            pltpu.VMEM((_BPW,), jnp.int32),
            pltpu.VMEM((_BPW, EMB), jnp.float32),
            pltpu.SemaphoreType.DMA,
        ],
        compiler_params=pltpu.CompilerParams(use_tc_tiling_on_sc=False),
    )
    def _sc_gather(idx_hbm, table_hbm, out_hbm, idx_v, rows_v, sem):
        wid = lax.axis_index("s") * _NC + lax.axis_index("c")
        base = wid * _BPW
        pltpu.sync_copy(idx_hbm.at[pl.ds(base, _BPW)], idx_v)
        pltpu.async_copy(table_hbm.at[idx_v], rows_v, sem).wait()
        pltpu.sync_copy(rows_v, out_hbm.at[pl.ds(base, _BPW)])

    return _sc_gather


VT = 2048
NV = -(-VOCAB // VT)


def _tc_body(e_ref, w1_ref, b1_ref, w2_ref, b2_ref, out_ref, h_ref, m_ref, s_ref):
    phase = pl.program_id(0)
    v = pl.program_id(1)

    @pl.when(jnp.logical_and(phase == 0, v == 0))
    def _init():
        h = jnp.dot(e_ref[...], w1_ref[...], preferred_element_type=jnp.float32)
        h_ref[...] = jnp.maximum(h + b1_ref[...], 0.0).astype(jnp.bfloat16)
        m_ref[...] = jnp.full_like(m_ref, -jnp.inf)
        s_ref[...] = jnp.zeros_like(s_ref)

    logits = (
        jnp.dot(h_ref[...], w2_ref[...], preferred_element_type=jnp.float32)
        + b2_ref[...]
    )

    @pl.when(phase == 0)
    def _pass_stats():
        cols = v * VT + lax.broadcasted_iota(jnp.int32, (1, VT), 1)
        lm = jnp.where(cols < VOCAB, logits, -jnp.inf)
        m_new = jnp.maximum(m_ref[...], jnp.max(lm, axis=1, keepdims=True))
        s_ref[...] = s_ref[...] * jnp.exp(m_ref[...] - m_new) + jnp.sum(
            jnp.exp(lm - m_new), axis=1, keepdims=True
        )
        m_ref[...] = m_new

    @pl.when(phase == 1)
    def _pass_write():
        out_ref[...] = logits - m_ref[...] - jnp.log(s_ref[...])


def _mlp_logsoftmax(e, W1, b1, W2, b2, interpret=False):
    return pl.pallas_call(
        _tc_body,
        grid=(2, NV),
        in_specs=[
            pl.BlockSpec((BATCH, 2 * CONTEXT * EMB), lambda p, v: (0, 0)),
            pl.BlockSpec((2 * CONTEXT * EMB, HIDDEN), lambda p, v: (0, 0)),
            pl.BlockSpec((1, HIDDEN), lambda p, v: (0, 0)),
            pl.BlockSpec((HIDDEN, VT), lambda p, v: (0, v)),
            pl.BlockSpec((1, VT), lambda p, v: (0, v)),
        ],
        out_specs=pl.BlockSpec(
            (BATCH, VT), lambda p, v: (0, jnp.where(p == 1, v, 0))
        ),
        out_shape=jax.ShapeDtypeStruct((BATCH, VOCAB), jnp.float32),
        scratch_shapes=[
            pltpu.VMEM((BATCH, HIDDEN), jnp.bfloat16),
            pltpu.VMEM((BATCH, 1), jnp.float32),
            pltpu.VMEM((BATCH, 1), jnp.float32),
        ],
        interpret=interpret,
    )(e, W1, b1, W2, b2)


def kernel(inputs, embeds, W1, b1, W2, b2):
    idx = inputs.reshape(-1).astype(jnp.int32)
    gathered = _sc_gather_fn()(idx, embeds)
    e = gathered.reshape(BATCH, 2 * CONTEXT * EMB)
    return _mlp_logsoftmax(
        e, W1, b1.reshape(1, HIDDEN), W2.astype(jnp.bfloat16), b2.reshape(1, VOCAB)
    )


# speedup vs baseline: 1.0361x; 1.0361x over previous
import functools

import jax
import jax.numpy as jnp
from jax import lax
from jax.experimental import pallas as pl
from jax.experimental.pallas import tpu as pltpu
from jax.experimental.pallas import tpu_sc as plsc

VOCAB = 100000
CONTEXT = 5
EMB = 64
BATCH = 1024
HIDDEN = 128
NLOOK = BATCH * 2 * CONTEXT
IN_FEAT = 2 * CONTEXT * EMB


_NC = 2
_NS = 16
_NW = _NC * _NS
_BPW = NLOOK // _NW


@functools.cache
def _sc_gather_fn():
    mesh = plsc.VectorSubcoreMesh(core_axis_name="c", subcore_axis_name="s")

    @functools.partial(
        pl.kernel,
        mesh=mesh,
        out_type=jax.ShapeDtypeStruct((NLOOK, EMB), jnp.float32),
        scratch_types=[
            pltpu.VMEM((_BPW,), jnp.int32),
            pltpu.VMEM((_BPW, EMB), jnp.float32),
            pltpu.SemaphoreType.DMA,
        ],
        compiler_params=pltpu.CompilerParams(use_tc_tiling_on_sc=False),
    )
    def _sc_gather(idx_hbm, table_hbm, out_hbm, idx_v, rows_v, sem):
        wid = lax.axis_index("s") * _NC + lax.axis_index("c")
        base = wid * _BPW
        pltpu.sync_copy(idx_hbm.at[pl.ds(base, _BPW)], idx_v)
        pltpu.async_copy(table_hbm.at[idx_v], rows_v, sem).wait()
        pltpu.sync_copy(rows_v, out_hbm.at[pl.ds(base, _BPW)])

    return _sc_gather


VT = 2048
NV = -(-VOCAB // VT)


def _stats_body(e_ref, w1_ref, b1_ref, w2_ref, b2_ref, h_ref, m_ref, s_ref):
    v = pl.program_id(0)

    @pl.when(v == 0)
    def _init():
        h = jnp.dot(e_ref[...], w1_ref[...], preferred_element_type=jnp.float32)
        h_ref[...] = jnp.maximum(h + b1_ref[...], 0.0).astype(jnp.bfloat16)
        m_ref[...] = jnp.full_like(m_ref, -jnp.inf)
        s_ref[...] = jnp.zeros_like(s_ref)

    logits = (
        jnp.dot(h_ref[...], w2_ref[...], preferred_element_type=jnp.float32)
        + b2_ref[...]
    )
    cols = v * VT + lax.broadcasted_iota(jnp.int32, (1, VT), 1)
    lm = jnp.where(cols < VOCAB, logits, -jnp.inf)
    m_new = jnp.maximum(m_ref[...], jnp.max(lm, axis=1, keepdims=True))
    s_ref[...] = s_ref[...] * jnp.exp(m_ref[...] - m_new) + jnp.sum(
        jnp.exp(lm - m_new), axis=1, keepdims=True
    )
    m_ref[...] = m_new


def _write_body(h_ref, w2_ref, b2_ref, c_ref, out_ref):
    out_ref[...] = (
        jnp.dot(h_ref[...], w2_ref[...], preferred_element_type=jnp.float32)
        + b2_ref[...]
    ) - c_ref[...]


def _mlp_logsoftmax(e, W1, b1, W2, b2):
    h, m, s = pl.pallas_call(
        _stats_body,
        grid=(NV,),
        in_specs=[
            pl.BlockSpec((BATCH, IN_FEAT), lambda v: (0, 0)),
            pl.BlockSpec((IN_FEAT, HIDDEN), lambda v: (0, 0)),
            pl.BlockSpec((1, HIDDEN), lambda v: (0, 0)),
            pl.BlockSpec((HIDDEN, VT), lambda v: (0, v)),
            pl.BlockSpec((1, VT), lambda v: (0, v)),
        ],
        out_specs=[
            pl.BlockSpec((BATCH, HIDDEN), lambda v: (0, 0)),
            pl.BlockSpec((BATCH, 1), lambda v: (0, 0)),
            pl.BlockSpec((BATCH, 1), lambda v: (0, 0)),
        ],
        out_shape=[
            jax.ShapeDtypeStruct((BATCH, HIDDEN), jnp.bfloat16),
            jax.ShapeDtypeStruct((BATCH, 1), jnp.float32),
            jax.ShapeDtypeStruct((BATCH, 1), jnp.float32),
        ],
    )(e, W1, b1, W2, b2)

    c = m + jnp.log(s)

    return pl.pallas_call(
        _write_body,
        grid=(NV,),
        in_specs=[
            pl.BlockSpec((BATCH, HIDDEN), lambda v: (0, 0)),
            pl.BlockSpec((HIDDEN, VT), lambda v: (0, v)),
            pl.BlockSpec((1, VT), lambda v: (0, v)),
            pl.BlockSpec((BATCH, 1), lambda v: (0, 0)),
        ],
        out_specs=pl.BlockSpec((BATCH, VT), lambda v: (0, v)),
        out_shape=jax.ShapeDtypeStruct((BATCH, VOCAB), jnp.float32),
    )(h, W2, b2, c)


def kernel(inputs, embeds, W1, b1, W2, b2):
    idx = inputs.reshape(-1).astype(jnp.int32)
    gathered = _sc_gather_fn()(idx, embeds)
    e = gathered.reshape(BATCH, IN_FEAT)
    return _mlp_logsoftmax(
        e, W1, b1.reshape(1, HIDDEN), W2.astype(jnp.bfloat16), b2.reshape(1, VOCAB)
    )


# speedup vs baseline: 1.0364x; 1.0003x over previous
import functools

import jax
import jax.numpy as jnp
from jax import lax
from jax.experimental import pallas as pl
from jax.experimental.pallas import tpu as pltpu
from jax.experimental.pallas import tpu_sc as plsc

VOCAB = 100000
CONTEXT = 5
EMB = 64
BATCH = 1024
HIDDEN = 128
NLOOK = BATCH * 2 * CONTEXT
IN_FEAT = 2 * CONTEXT * EMB


_NC = 2
_NS = 16
_NW = _NC * _NS
_BPW = NLOOK // _NW


@functools.cache
def _sc_gather_fn():
    mesh = plsc.VectorSubcoreMesh(core_axis_name="c", subcore_axis_name="s")

    @functools.partial(
        pl.kernel,
        mesh=mesh,
        out_type=jax.ShapeDtypeStruct((NLOOK, EMB), jnp.float32),
        scratch_types=[
            pltpu.VMEM((_BPW,), jnp.int32),
            pltpu.VMEM((_BPW, EMB), jnp.float32),
            pltpu.SemaphoreType.DMA,
        ],
        compiler_params=pltpu.CompilerParams(use_tc_tiling_on_sc=False),
    )
    def _sc_gather(idx_hbm, table_hbm, out_hbm, idx_v, rows_v, sem):
        wid = lax.axis_index("s") * _NC + lax.axis_index("c")
        base = wid * _BPW
        pltpu.sync_copy(idx_hbm.at[pl.ds(base, _BPW)], idx_v)
        pltpu.async_copy(table_hbm.at[idx_v], rows_v, sem).wait()
        pltpu.sync_copy(rows_v, out_hbm.at[pl.ds(base, _BPW)])

    return _sc_gather


VT = 2048
NV = -(-VOCAB // VT)
TAIL = VOCAB - (NV - 1) * VT
_WQ = 4


def _stats_body(e_ref, w1_ref, b1_ref, w2_ref, b2_ref, h_ref, m_ref, s_ref):
    v = pl.program_id(0)

    @pl.when(v == 0)
    def _init():
        h = jnp.dot(e_ref[...], w1_ref[...], preferred_element_type=jnp.float32)
        h_ref[...] = jnp.maximum(h + b1_ref[...], 0.0).astype(jnp.bfloat16)
        m_ref[...] = jnp.full_like(m_ref, -jnp.inf)
        s_ref[...] = jnp.zeros_like(s_ref)

    logits = (
        jnp.dot(h_ref[...], w2_ref[...], preferred_element_type=jnp.float32)
        + b2_ref[...]
    )
    cols = v * VT + lax.broadcasted_iota(jnp.int32, (1, VT), 1)
    lm = jnp.where(cols < VOCAB, logits, -jnp.inf)
    m_new = jnp.maximum(m_ref[...], jnp.max(lm, axis=1, keepdims=True))
    s_ref[...] = s_ref[...] * jnp.exp(m_ref[...] - m_new) + jnp.sum(
        jnp.exp(lm - m_new), axis=1, keepdims=True
    )
    m_ref[...] = m_new


def _write_body(h_ref, w2_ref, b2_ref, c_ref, out_hbm, buf, tail_buf, sems):
    v = pl.program_id(0)
    slot = lax.rem(v, _WQ)

    x = (
        jnp.dot(h_ref[...], w2_ref[...], preferred_element_type=jnp.float32)
        + b2_ref[...]
    ) - c_ref[...]

    @pl.when(v >= _WQ)
    def _reclaim():
        pltpu.make_async_copy(
            buf.at[slot], out_hbm.at[:, pl.ds((v - _WQ) * VT, VT)], sems.at[slot]
        ).wait()

    @pl.when(v < NV - 1)
    def _fire():
        buf[slot] = x
        pltpu.make_async_copy(
            buf.at[slot], out_hbm.at[:, pl.ds(v * VT, VT)], sems.at[slot]
        ).start()

    @pl.when(v == NV - 1)
    def _fire_tail_and_drain():
        tail_buf[...] = x[:, :TAIL]
        pltpu.make_async_copy(
            tail_buf,
            out_hbm.at[:, pl.ds((NV - 1) * VT, TAIL)],
            sems.at[slot],
        ).start()
        for j in range(1, _WQ):
            sl = (NV - 1 - j) % _WQ
            pltpu.make_async_copy(
                buf.at[sl],
                out_hbm.at[:, pl.ds((NV - 1 - j) * VT, VT)],
                sems.at[sl],
            ).wait()
        pltpu.make_async_copy(
            tail_buf,
            out_hbm.at[:, pl.ds((NV - 1) * VT, TAIL)],
            sems.at[slot],
        ).wait()


def _mlp_logsoftmax(e, W1, b1, W2, b2):
    h, m, s = pl.pallas_call(
        _stats_body,
        grid=(NV,),
        in_specs=[
            pl.BlockSpec((BATCH, IN_FEAT), lambda v: (0, 0)),
            pl.BlockSpec((IN_FEAT, HIDDEN), lambda v: (0, 0)),
            pl.BlockSpec((1, HIDDEN), lambda v: (0, 0)),
            pl.BlockSpec((HIDDEN, VT), lambda v: (0, v)),
            pl.BlockSpec((1, VT), lambda v: (0, v)),
        ],
        out_specs=[
            pl.BlockSpec((BATCH, HIDDEN), lambda v: (0, 0)),
            pl.BlockSpec((BATCH, 1), lambda v: (0, 0)),
            pl.BlockSpec((BATCH, 1), lambda v: (0, 0)),
        ],
        out_shape=[
            jax.ShapeDtypeStruct((BATCH, HIDDEN), jnp.bfloat16),
            jax.ShapeDtypeStruct((BATCH, 1), jnp.float32),
            jax.ShapeDtypeStruct((BATCH, 1), jnp.float32),
        ],
    )(e, W1, b1, W2, b2)

    c = m + jnp.log(s)

    return pl.pallas_call(
        _write_body,
        grid=(NV,),
        in_specs=[
            pl.BlockSpec((BATCH, HIDDEN), lambda v: (0, 0)),
            pl.BlockSpec((HIDDEN, VT), lambda v: (0, v)),
            pl.BlockSpec((1, VT), lambda v: (0, v)),
            pl.BlockSpec((BATCH, 1), lambda v: (0, 0)),
        ],
        out_specs=pl.BlockSpec(memory_space=pl.ANY),
        out_shape=jax.ShapeDtypeStruct((BATCH, VOCAB), jnp.float32),
        scratch_shapes=[
            pltpu.VMEM((_WQ, BATCH, VT), jnp.float32),
            pltpu.VMEM((BATCH, TAIL), jnp.float32),
            pltpu.SemaphoreType.DMA((_WQ,)),
        ],
    )(h, W2, b2, c)


def kernel(inputs, embeds, W1, b1, W2, b2):
    idx = inputs.reshape(-1).astype(jnp.int32)
    gathered = _sc_gather_fn()(idx, embeds)
    e = gathered.reshape(BATCH, IN_FEAT)
    return _mlp_logsoftmax(
        e, W1, b1.reshape(1, HIDDEN), W2.astype(jnp.bfloat16), b2.reshape(1, VOCAB)
    )


# speedup vs baseline: 1.0894x; 1.0511x over previous
import functools

import jax
import jax.numpy as jnp
from jax import lax
from jax.experimental import pallas as pl
from jax.experimental.pallas import tpu as pltpu
from jax.experimental.pallas import tpu_sc as plsc

VOCAB = 100000
CONTEXT = 5
EMB = 64
BATCH = 1024
HIDDEN = 128
NLOOK = BATCH * 2 * CONTEXT
IN_FEAT = 2 * CONTEXT * EMB


_NC = 2
_NS = 16
_NW = _NC * _NS
_BPW = NLOOK // _NW


@functools.cache
def _sc_gather_fn():
    mesh = plsc.VectorSubcoreMesh(core_axis_name="c", subcore_axis_name="s")

    @functools.partial(
        pl.kernel,
        mesh=mesh,
        out_type=jax.ShapeDtypeStruct((NLOOK, EMB), jnp.float32),
        scratch_types=[
            pltpu.VMEM((_BPW,), jnp.int32),
            pltpu.VMEM((_BPW, EMB), jnp.float32),
            pltpu.SemaphoreType.DMA,
        ],
        compiler_params=pltpu.CompilerParams(use_tc_tiling_on_sc=False),
    )
    def _sc_gather(idx_hbm, table_hbm, out_hbm, idx_v, rows_v, sem):
        wid = lax.axis_index("s") * _NC + lax.axis_index("c")
        base = wid * _BPW
        pltpu.sync_copy(idx_hbm.at[pl.ds(base, _BPW)], idx_v)
        pltpu.async_copy(table_hbm.at[idx_v], rows_v, sem).wait()
        pltpu.sync_copy(rows_v, out_hbm.at[pl.ds(base, _BPW)])

    return _sc_gather


BS = 32
NB = BATCH // BS


def _hidden_body(e_ref, w1_ref, b1_ref, h_ref):
    hh = jnp.dot(e_ref[...], w1_ref[...], preferred_element_type=jnp.float32)
    h_ref[...] = jnp.maximum(hh + b1_ref[...], 0.0).astype(jnp.bfloat16)


def _fused_body(h_ref, w2_ref, b2_ref, out_ref):
    i = pl.program_id(0)
    hs = h_ref[pl.ds(i * BS, BS), :]
    out_ref[...] = (
        jnp.dot(hs, w2_ref[...], preferred_element_type=jnp.float32)
        + b2_ref[...]
    )
    x = out_ref[...]
    m = jnp.max(x, axis=1, keepdims=True)
    s = jnp.sum(jnp.exp(x - m), axis=1, keepdims=True)
    out_ref[...] = x - (m + jnp.log(s))


def _mlp_logsoftmax(e, W1, b1, W2, b2):
    h = pl.pallas_call(
        _hidden_body,
        out_shape=jax.ShapeDtypeStruct((BATCH, HIDDEN), jnp.bfloat16),
    )(e, W1, b1)

    return pl.pallas_call(
        _fused_body,
        grid=(NB,),
        in_specs=[
            pl.BlockSpec((BATCH, HIDDEN), lambda i: (0, 0)),
            pl.BlockSpec((HIDDEN, VOCAB), lambda i: (0, 0)),
            pl.BlockSpec((1, VOCAB), lambda i: (0, 0)),
        ],
        out_specs=pl.BlockSpec((BS, VOCAB), lambda i: (i, 0)),
        out_shape=jax.ShapeDtypeStruct((BATCH, VOCAB), jnp.float32),
        compiler_params=pltpu.CompilerParams(
            vmem_limit_bytes=127 * 1024 * 1024,
        ),
    )(h, W2, b2)


def kernel(inputs, embeds, W1, b1, W2, b2):
    idx = inputs.reshape(-1).astype(jnp.int32)
    gathered = _sc_gather_fn()(idx, embeds)
    e = gathered.reshape(BATCH, IN_FEAT)
    return _mlp_logsoftmax(
        e, W1, b1.reshape(1, HIDDEN), W2.astype(jnp.bfloat16), b2.reshape(1, VOCAB)
    )
